# baseline (speedup 1.0000x reference)
import functools

import jax
import jax.numpy as jnp
from jax import lax
from jax.experimental import pallas as pl
from jax.experimental.pallas import tpu as pltpu
from jax.experimental.pallas import tpu_sc as plsc

N_NODES = 10000
N_EDGES = 320000
D = 128

NW = 32
NPT = 320
NPAD = NW * NPT
ECHUNK = 4000
NGROUPS = ECHUNK // 16
NCHUNKS = N_EDGES // ECHUNK
GB = 64
NBUF = 6
NB_ROWS = (ECHUNK + 128 + GB - 1) // GB
ACC_ROWS = NPT + 16


def _mm_body(f_ref, wt_ref, wp_ref, bt_ref, bp_ref, q_ref, c_ref):
    f = f_ref[...]
    wt = wt_ref[...]
    q_ref[...] = jnp.dot(f, wt, preferred_element_type=jnp.float32)
    c_ref[...] = (
        jnp.dot(f, wt + wp_ref[...], preferred_element_type=jnp.float32)
        + bt_ref[...] + bp_ref[...]
    )


def _matmuls(featp, W_theta, b_theta, W_phi, b_phi):
    blk = 1024
    grid = NPAD // blk
    return pl.pallas_call(
        _mm_body,
        grid=(grid,),
        in_specs=[
            pl.BlockSpec((blk, D), lambda i: (i, 0)),
            pl.BlockSpec((D, D), lambda i: (0, 0)),
            pl.BlockSpec((D, D), lambda i: (0, 0)),
            pl.BlockSpec((1, D), lambda i: (0, 0)),
            pl.BlockSpec((1, D), lambda i: (0, 0)),
        ],
        out_specs=[
            pl.BlockSpec((blk, D), lambda i: (i, 0)),
            pl.BlockSpec((blk, D), lambda i: (i, 0)),
        ],
        out_shape=[
            jax.ShapeDtypeStruct((NPAD, D), jnp.float32),
            jax.ShapeDtypeStruct((NPAD, D), jnp.float32),
        ],
    )(featp, W_theta, W_phi, b_theta.reshape(1, D), b_phi.reshape(1, D))


def _sub_body(c_ref, m_ref, o_ref):
    o_ref[...] = c_ref[...] - m_ref[...]


def _subtract(C, M):
    blk = 1024
    return pl.pallas_call(
        _sub_body,
        grid=(NPAD // blk,),
        in_specs=[
            pl.BlockSpec((blk, D), lambda i: (i, 0)),
            pl.BlockSpec((blk, D), lambda i: (i, 0)),
        ],
        out_specs=pl.BlockSpec((blk, D), lambda i: (i, 0)),
        out_shape=jax.ShapeDtypeStruct((NPAD, D), jnp.float32),
    )(C, M)


def _segmin_sc(Q, edges):
    mesh = plsc.VectorSubcoreMesh(core_axis_name="c", subcore_axis_name="s")

    @functools.partial(
        pl.kernel,
        mesh=mesh,
        out_type=jax.ShapeDtypeStruct((NPAD * D,), jnp.float32),
        scratch_types=[
            pltpu.VMEM((ECHUNK,), jnp.int32),
            pltpu.VMEM((NB_ROWS, GB), jnp.int32),
            pltpu.VMEM((NB_ROWS * GB,), jnp.int32),
            [pltpu.VMEM((GB, D), jnp.float32) for _ in range(NBUF)],
            pltpu.VMEM((ACC_ROWS * D,), jnp.float32),
            [pltpu.SemaphoreType.DMA for _ in range(NBUF)],
        ],
        compiler_params=pltpu.CompilerParams(needs_layout_passes=False),
    )
    def seg_min(q_hbm, edges_hbm, m_hbm,
                ebuf, csrc, cdst, rows, acc, sems):
        wid = lax.axis_index("c") * 16 + lax.axis_index("s")
        base = wid * NPT

        inf16 = jnp.full((16,), jnp.inf, jnp.float32)

        def init_body(i, _):
            acc[pl.ds(i * 16, 16)] = inf16
            return 0
        lax.fori_loop(0, (ACC_ROWS * D) // 16, init_body, 0)

        iot = lax.iota(jnp.int32, 16)
        z16 = jnp.zeros((16,), jnp.int32)
        s16 = jnp.full((16,), NPT, jnp.int32)
        _dnums = lax.GatherDimensionNumbers(
            offset_dims=(), collapsed_slice_dims=(0,), start_index_map=(0,))

        def _vgather(x, idx):
            return lax.gather(
                x, idx[:, None], _dnums, (1,),
                indices_are_sorted=False, unique_indices=False,
                mode=lax.GatherScatterMode.PROMISE_IN_BOUNDS)

        _sh_idx = [jnp.maximum(iot - k, 0) for k in (1, 2, 4, 8)]
        _lane15 = jnp.full((16,), 15, jnp.int32)

        def chunk_body(i, _):
            off0 = pl.multiple_of(i * ECHUNK, 8)
            pltpu.sync_copy(edges_hbm.at[pl.ds(off0, ECHUNK)], ebuf)

            def scan_group(g, off_v):
                ev = ebuf[pl.ds(g * 16, 16)]
                loc = (ev & 0xFFFF) - base
                m = (loc >= 0) & (loc < NPT)
                svec = ev >> 16
                mi = m.astype(jnp.int32)
                x = mi
                for k, si in zip((1, 2, 4, 8), _sh_idx):
                    x = x + jnp.where(iot >= k, _vgather(x, si), 0)
                pos = (x - mi) + off_v
                plsc.store_scatter(csrc, [pos // GB, pos % GB], svec, mask=m)
                plsc.store_scatter(cdst, [pos], loc, mask=m)
                return off_v + _vgather(x, _lane15)

            off_v = lax.fori_loop(0, NGROUPS, scan_group,
                                  jnp.zeros((16,), jnp.int32))

            for t in range(8):
                pv = off_v + (t * 16) + iot
                plsc.store_scatter(csrc, [pv // GB, pv % GB], z16)
                plsc.store_scatter(cdst, [pv], s16)
            off = off_v[0]
            off_r = (off + 15) & (-16)

            nb = (off_r + GB - 1) // GB

            def round_body(rr, _):
                jb = rr * NBUF
                for b in range(NBUF):
                    @pl.when(jb + b < nb)
                    def _start(b=b):
                        pltpu.make_async_copy(
                            q_hbm.at[csrc.at[jb + b]], rows[b],
                            sems[b]).start()
                for b in range(NBUF):
                    @pl.when(jb + b < nb)
                    def _drain(b=b):
                        j = jb + b
                        pltpu.make_async_copy(
                            q_hbm.at[csrc.at[j]], rows[b], sems[b]).wait()
                        ng = jnp.minimum(
                            GB // 16, (off_r - j * GB + 15) // 16)

                        def gbody(g, _3, b=b, j=j):
                            dvec = cdst[pl.ds(j * GB + g * 16, 16)]
                            for lane in range(16):
                                d = dvec[lane]
                                ab = d * D
                                rrow = rows[b].at[g * 16 + lane]

                                def cbody(c, _2, ab=ab, rrow=rrow):
                                    a = acc[pl.ds(ab + c * 16, 16)]
                                    r = rrow[pl.ds(c * 16, 16)]
                                    acc[pl.ds(ab + c * 16, 16)] = (
                                        jnp.minimum(a, r))
                                    return 0
                                lax.fori_loop(0, D // 16, cbody, 0)
                            return 0

                        lax.fori_loop(0, ng, gbody, 0)
                return 0

            lax.fori_loop(0, (nb + NBUF - 1) // NBUF, round_body, 0)
            return 0

        lax.fori_loop(0, NCHUNKS, chunk_body, 0)
        pltpu.sync_copy(acc.at[pl.ds(0, NPT * D)],
                        m_hbm.at[pl.ds(base * D, NPT * D)])

    return seg_min(Q, edges)


def kernel(feat, edge_index, W_theta, b_theta, W_phi, b_phi):
    feat = feat.astype(jnp.float32)
    src = edge_index[0].astype(jnp.int32)
    dst = edge_index[1].astype(jnp.int32)
    edges_packed = jnp.bitwise_or(jnp.left_shift(src, 16), dst)
    featp = jnp.pad(feat, ((0, NPAD - N_NODES), (0, 0)))
    Q, C = _matmuls(featp, W_theta, b_theta, W_phi, b_phi)
    M = _segmin_sc(Q, edges_packed).reshape(NPAD, D)
    out = _subtract(C, M)
    return out[:N_NODES]


# speedup vs baseline: 2.1232x; 2.1232x over previous
import functools

import jax
import jax.numpy as jnp
from jax import lax
from jax.experimental import pallas as pl
from jax.experimental.pallas import tpu as pltpu
from jax.experimental.pallas import tpu_sc as plsc

N_NODES = 10000
N_EDGES = 320000
D = 128

NW = 32
NPT = 320
NPAD = NW * NPT
ECHUNK = 10000
NGROUPS = ECHUNK // 16
NCHUNKS = N_EDGES // ECHUNK
GB = 64
NBUF = 5
NB_ROWS = (ECHUNK + 128 + GB - 1) // GB
ACC_ROWS = NPT + 16


def _mm_body(f_ref, wt_ref, wp_ref, bt_ref, bp_ref, q_ref, c_ref):
    f = f_ref[...]
    wt = wt_ref[...]
    q_ref[...] = jnp.dot(f, wt, preferred_element_type=jnp.float32)
    c_ref[...] = (
        jnp.dot(f, wt + wp_ref[...], preferred_element_type=jnp.float32)
        + bt_ref[...] + bp_ref[...]
    )


def _matmuls(featp, W_theta, b_theta, W_phi, b_phi):
    blk = 1024
    grid = NPAD // blk
    return pl.pallas_call(
        _mm_body,
        grid=(grid,),
        in_specs=[
            pl.BlockSpec((blk, D), lambda i: (i, 0)),
            pl.BlockSpec((D, D), lambda i: (0, 0)),
            pl.BlockSpec((D, D), lambda i: (0, 0)),
            pl.BlockSpec((1, D), lambda i: (0, 0)),
            pl.BlockSpec((1, D), lambda i: (0, 0)),
        ],
        out_specs=[
            pl.BlockSpec((blk, D), lambda i: (i, 0)),
            pl.BlockSpec((blk, D), lambda i: (i, 0)),
        ],
        out_shape=[
            jax.ShapeDtypeStruct((NPAD, D), jnp.float32),
            jax.ShapeDtypeStruct((NPAD, D), jnp.float32),
        ],
    )(featp, W_theta, W_phi, b_theta.reshape(1, D), b_phi.reshape(1, D))


def _sub_body(c_ref, m_ref, o_ref):
    o_ref[...] = c_ref[...] - m_ref[...]


def _subtract(C, M):
    blk = 1024
    return pl.pallas_call(
        _sub_body,
        grid=(NPAD // blk,),
        in_specs=[
            pl.BlockSpec((blk, D), lambda i: (i, 0)),
            pl.BlockSpec((blk, D), lambda i: (i, 0)),
        ],
        out_specs=pl.BlockSpec((blk, D), lambda i: (i, 0)),
        out_shape=jax.ShapeDtypeStruct((NPAD, D), jnp.float32),
    )(C, M)


def _segmin_sc(Q, edges):
    mesh = plsc.VectorSubcoreMesh(core_axis_name="c", subcore_axis_name="s")

    @functools.partial(
        pl.kernel,
        mesh=mesh,
        out_type=jax.ShapeDtypeStruct((NPAD * D,), jnp.float32),
        scratch_types=[
            pltpu.VMEM((ECHUNK,), jnp.int32),
            pltpu.VMEM((NB_ROWS, GB), jnp.int32),
            pltpu.VMEM((NB_ROWS * GB,), jnp.int32),
            [pltpu.VMEM((GB, D), jnp.float32) for _ in range(NBUF)],
            pltpu.VMEM((ACC_ROWS * D,), jnp.float32),
            [pltpu.SemaphoreType.DMA for _ in range(NBUF)],
        ],
        compiler_params=pltpu.CompilerParams(needs_layout_passes=False),
    )
    def seg_min(q_hbm, edges_hbm, m_hbm,
                ebuf, csrc, cdst, rows, acc, sems):
        wid = lax.axis_index("c") * 16 + lax.axis_index("s")
        base = wid * NPT

        inf16 = jnp.full((16,), jnp.inf, jnp.float32)

        def init_body(i, _):
            acc[pl.ds(i * 16, 16)] = inf16
            return 0
        lax.fori_loop(0, (ACC_ROWS * D) // 16, init_body, 0)

        iot = lax.iota(jnp.int32, 16)
        z16 = jnp.zeros((16,), jnp.int32)
        s16 = jnp.full((16,), NPT, jnp.int32)
        _dnums = lax.GatherDimensionNumbers(
            offset_dims=(), collapsed_slice_dims=(0,), start_index_map=(0,))

        def _vgather(x, idx):
            return lax.gather(
                x, idx[:, None], _dnums, (1,),
                indices_are_sorted=False, unique_indices=False,
                mode=lax.GatherScatterMode.PROMISE_IN_BOUNDS)

        _sh_idx = [jnp.maximum(iot - k, 0) for k in (1, 2, 4, 8)]
        _lane15 = jnp.full((16,), 15, jnp.int32)

        def chunk_body(i, _):
            off0 = pl.multiple_of(i * ECHUNK, 8)
            pltpu.sync_copy(edges_hbm.at[pl.ds(off0, ECHUNK)], ebuf)

            def scan_group(g, off_v):
                ev = ebuf[pl.ds(g * 16, 16)]
                loc = (ev & 0xFFFF) - base
                m = (loc >= 0) & (loc < NPT)
                svec = ev >> 16
                mi = m.astype(jnp.int32)
                x = mi
                for k, si in zip((1, 2, 4, 8), _sh_idx):
                    x = x + jnp.where(iot >= k, _vgather(x, si), 0)
                pos = (x - mi) + off_v
                plsc.store_scatter(csrc, [pos // GB, pos % GB], svec, mask=m)
                plsc.store_scatter(cdst, [pos], loc, mask=m)
                return off_v + _vgather(x, _lane15)

            off_v = lax.fori_loop(0, NGROUPS, scan_group,
                                  jnp.zeros((16,), jnp.int32))

            for t in range(8):
                pv = off_v + (t * 16) + iot
                plsc.store_scatter(csrc, [pv // GB, pv % GB], z16)
                plsc.store_scatter(cdst, [pv], s16)
            off = off_v[0]
            off_r = (off + 15) & (-16)

            nb = (off_r + GB - 1) // GB

            def round_body(rr, _):
                jb = rr * NBUF
                for b in range(NBUF):
                    @pl.when(jb + b < nb)
                    def _start(b=b):
                        pltpu.make_async_copy(
                            q_hbm.at[csrc.at[jb + b]], rows[b],
                            sems[b]).start()
                for b in range(NBUF):
                    @pl.when(jb + b < nb)
                    def _drain(b=b):
                        j = jb + b
                        pltpu.make_async_copy(
                            q_hbm.at[csrc.at[j]], rows[b], sems[b]).wait()
                        ng = jnp.minimum(
                            GB // 16, (off_r - j * GB + 15) // 16)

                        def gbody(g, _3, b=b, j=j):
                            dvec = cdst[pl.ds(j * GB + g * 16, 16)]
                            for lane in range(16):
                                d = dvec[lane]
                                ab = d * D
                                rrow = rows[b].at[g * 16 + lane]

                                def cbody(c, _2, ab=ab, rrow=rrow):
                                    a = acc[pl.ds(ab + c * 16, 16)]
                                    r = rrow[pl.ds(c * 16, 16)]
                                    acc[pl.ds(ab + c * 16, 16)] = (
                                        jnp.minimum(a, r))
                                    return 0
                                lax.fori_loop(0, D // 16, cbody, 0)
                            return 0

                        lax.fori_loop(0, ng, gbody, 0)
                return 0

            lax.fori_loop(0, (nb + NBUF - 1) // NBUF, round_body, 0)
            return 0

        lax.fori_loop(0, NCHUNKS, chunk_body, 0)
        pltpu.sync_copy(acc.at[pl.ds(0, NPT * D)],
                        m_hbm.at[pl.ds(base * D, NPT * D)])

    return seg_min(Q, edges)


def kernel(feat, edge_index, W_theta, b_theta, W_phi, b_phi):
    feat = feat.astype(jnp.float32)
    src = edge_index[0].astype(jnp.int32)
    dst = edge_index[1].astype(jnp.int32)
    edges_packed = jnp.bitwise_or(jnp.left_shift(src, 16), dst)
    featp = jnp.pad(feat, ((0, NPAD - N_NODES), (0, 0)))
    Q, C = _matmuls(featp, W_theta, b_theta, W_phi, b_phi)
    M = _segmin_sc(Q, edges_packed).reshape(NPAD, D)
    out = _subtract(C, M)
    return out[:N_NODES]


# speedup vs baseline: 2.7663x; 1.3029x over previous
import functools

import jax
import jax.numpy as jnp
from jax import lax
from jax.experimental import pallas as pl
from jax.experimental.pallas import tpu as pltpu
from jax.experimental.pallas import tpu_sc as plsc

N_NODES = 10000
N_EDGES = 320000
D = 128

NW = 32
NPT = 320
NPAD = NW * NPT
ECHUNK = 10000
NGROUPS = ECHUNK // 16
NCHUNKS = N_EDGES // ECHUNK
GB = 32
NBUF = 6
NB_ROWS = (ECHUNK + 128 + GB - 1) // GB
ACC_ROWS = NPT + 16


def _mm_body(f_ref, wt_ref, wp_ref, bt_ref, bp_ref, q_ref, c_ref):
    f = f_ref[...]
    wt = wt_ref[...]
    q_ref[...] = jnp.dot(f, wt, preferred_element_type=jnp.float32)
    c_ref[...] = (
        jnp.dot(f, wt + wp_ref[...], preferred_element_type=jnp.float32)
        + bt_ref[...] + bp_ref[...]
    )


def _matmuls(featp, W_theta, b_theta, W_phi, b_phi):
    blk = 1024
    grid = NPAD // blk
    return pl.pallas_call(
        _mm_body,
        grid=(grid,),
        in_specs=[
            pl.BlockSpec((blk, D), lambda i: (i, 0)),
            pl.BlockSpec((D, D), lambda i: (0, 0)),
            pl.BlockSpec((D, D), lambda i: (0, 0)),
            pl.BlockSpec((1, D), lambda i: (0, 0)),
            pl.BlockSpec((1, D), lambda i: (0, 0)),
        ],
        out_specs=[
            pl.BlockSpec((blk, D), lambda i: (i, 0)),
            pl.BlockSpec((blk, D), lambda i: (i, 0)),
        ],
        out_shape=[
            jax.ShapeDtypeStruct((NPAD, D), jnp.float32),
            jax.ShapeDtypeStruct((NPAD, D), jnp.float32),
        ],
    )(featp, W_theta, W_phi, b_theta.reshape(1, D), b_phi.reshape(1, D))


def _sub_body(c_ref, m_ref, o_ref):
    o_ref[...] = c_ref[...] - m_ref[...]


def _subtract(C, M):
    blk = 1024
    return pl.pallas_call(
        _sub_body,
        grid=(NPAD // blk,),
        in_specs=[
            pl.BlockSpec((blk, D), lambda i: (i, 0)),
            pl.BlockSpec((blk, D), lambda i: (i, 0)),
        ],
        out_specs=pl.BlockSpec((blk, D), lambda i: (i, 0)),
        out_shape=jax.ShapeDtypeStruct((NPAD, D), jnp.float32),
    )(C, M)


def _segmin_sc(Q, edges):
    mesh = plsc.VectorSubcoreMesh(core_axis_name="c", subcore_axis_name="s")

    @functools.partial(
        pl.kernel,
        mesh=mesh,
        out_type=jax.ShapeDtypeStruct((NPAD * D,), jnp.float32),
        scratch_types=[
            pltpu.VMEM((ECHUNK,), jnp.int32),
            pltpu.VMEM((NB_ROWS, GB), jnp.int32),
            pltpu.VMEM((NB_ROWS * GB,), jnp.int32),
            [pltpu.VMEM((GB, D), jnp.float32) for _ in range(NBUF)],
            pltpu.VMEM((ACC_ROWS * D,), jnp.float32),
            [pltpu.SemaphoreType.DMA for _ in range(NBUF)],
        ],
        compiler_params=pltpu.CompilerParams(needs_layout_passes=False),
    )
    def seg_min(q_hbm, edges_hbm, m_hbm,
                ebuf, csrc, cdst, rows, acc, sems):
        wid = lax.axis_index("c") * 16 + lax.axis_index("s")
        base = wid * NPT

        inf16 = jnp.full((16,), jnp.inf, jnp.float32)

        def init_body(i, _):
            acc[pl.ds(i * 16, 16)] = inf16
            return 0
        lax.fori_loop(0, (ACC_ROWS * D) // 16, init_body, 0)

        iot = lax.iota(jnp.int32, 16)
        z16 = jnp.zeros((16,), jnp.int32)
        s16 = jnp.full((16,), NPT, jnp.int32)
        _dnums = lax.GatherDimensionNumbers(
            offset_dims=(), collapsed_slice_dims=(0,), start_index_map=(0,))

        def _vgather(x, idx):
            return lax.gather(
                x, idx[:, None], _dnums, (1,),
                indices_are_sorted=False, unique_indices=False,
                mode=lax.GatherScatterMode.PROMISE_IN_BOUNDS)

        _sh_idx = [jnp.maximum(iot - k, 0) for k in (1, 2, 4, 8)]
        _lane15 = jnp.full((16,), 15, jnp.int32)

        def chunk_body(i, _):
            off0 = pl.multiple_of(i * ECHUNK, 8)
            pltpu.sync_copy(edges_hbm.at[pl.ds(off0, ECHUNK)], ebuf)

            def scan_group(g, off_v):
                ev = ebuf[pl.ds(g * 16, 16)]
                loc = (ev & 0xFFFF) - base
                m = (loc >= 0) & (loc < NPT)
                svec = ev >> 16
                mi = m.astype(jnp.int32)
                x = mi
                for k, si in zip((1, 2, 4, 8), _sh_idx):
                    x = x + jnp.where(iot >= k, _vgather(x, si), 0)
                pos = (x - mi) + off_v
                plsc.store_scatter(csrc, [pos // GB, pos % GB], svec, mask=m)
                plsc.store_scatter(cdst, [pos], loc, mask=m)
                return off_v + _vgather(x, _lane15)

            off_v = lax.fori_loop(0, NGROUPS, scan_group,
                                  jnp.zeros((16,), jnp.int32))

            for t in range(8):
                pv = off_v + (t * 16) + iot
                plsc.store_scatter(csrc, [pv // GB, pv % GB], z16)
                plsc.store_scatter(cdst, [pv], s16)
            off = off_v[0]
            off_r = (off + 15) & (-16)

            nb = (off_r + GB - 1) // GB

            def round_body(rr, _):
                jb = rr * NBUF
                for b in range(NBUF):
                    @pl.when(jb + b < nb)
                    def _start(b=b):
                        pltpu.make_async_copy(
                            q_hbm.at[csrc.at[jb + b]], rows[b],
                            sems[b]).start()
                for b in range(NBUF):
                    @pl.when(jb + b < nb)
                    def _drain(b=b):
                        j = jb + b
                        pltpu.make_async_copy(
                            q_hbm.at[csrc.at[j]], rows[b], sems[b]).wait()
                        ng = jnp.minimum(
                            GB // 16, (off_r - j * GB + 15) // 16)

                        def gbody(g, _3, b=b, j=j):
                            dvec = cdst[pl.ds(j * GB + g * 16, 16)]
                            for lane in range(16):
                                d = dvec[lane]
                                ab = d * D
                                rrow = rows[b].at[g * 16 + lane]

                                def cbody(c, _2, ab=ab, rrow=rrow):
                                    a = acc[pl.ds(ab + c * 16, 16)]
                                    r = rrow[pl.ds(c * 16, 16)]
                                    acc[pl.ds(ab + c * 16, 16)] = (
                                        jnp.minimum(a, r))
                                    return 0
                                lax.fori_loop(0, D // 16, cbody, 0)
                            return 0

                        lax.fori_loop(0, ng, gbody, 0)
                return 0

            lax.fori_loop(0, (nb + NBUF - 1) // NBUF, round_body, 0)
            return 0

        lax.fori_loop(0, NCHUNKS, chunk_body, 0)
        pltpu.sync_copy(acc.at[pl.ds(0, NPT * D)],
                        m_hbm.at[pl.ds(base * D, NPT * D)])

    return seg_min(Q, edges)


def kernel(feat, edge_index, W_theta, b_theta, W_phi, b_phi):
    feat = feat.astype(jnp.float32)
    src = edge_index[0].astype(jnp.int32)
    dst = edge_index[1].astype(jnp.int32)
    edges_packed = jnp.bitwise_or(jnp.left_shift(src, 16), dst)
    featp = jnp.pad(feat, ((0, NPAD - N_NODES), (0, 0)))
    Q, C = _matmuls(featp, W_theta, b_theta, W_phi, b_phi)
    M = _segmin_sc(Q, edges_packed).reshape(NPAD, D)
    out = _subtract(C, M)
    return out[:N_NODES]
